# Initial kernel scaffold; baseline (speedup 1.0000x reference)
import jax
import jax.numpy as jnp
from jax.experimental import pallas as pl


def kernel(feature, edge_index, W, b):
    raise NotImplementedError("write your pallas kernel here")


# speedup vs baseline: 1.7285x; 1.7285x over previous
import functools

import jax
import jax.numpy as jnp
from jax import lax
from jax.experimental import pallas as pl
from jax.experimental.pallas import tpu as pltpu
from jax.experimental.pallas import tpu_sc as plsc

NC = 2
NS = 16
CHUNK = 128
D = 128
G = 8
DG = D // G
WB = 320


def _sc_aggregate(feat16, sidx8, dst, zeros, ones, *, npad, cw):
    rpt = npad // NS
    gpc = G // NC
    mesh = plsc.VectorSubcoreMesh(
        core_axis_name="c", subcore_axis_name="s", num_cores=NC, num_subcores=NS
    )

    @functools.partial(
        pl.kernel,
        out_type=(
            jax.ShapeDtypeStruct((npad, D), jnp.float32),
            jax.ShapeDtypeStruct((npad, DG), jnp.float32),
        ),
        mesh=mesh,
        compiler_params=pltpu.CompilerParams(use_tc_tiling_on_sc=False),
        scratch_types=[
            pltpu.VMEM((CHUNK,), jnp.int32),
            pltpu.VMEM((CHUNK,), jnp.int32),
            pltpu.VMEM((CHUNK, DG), jnp.float32),
            pltpu.VMEM((CHUNK, DG), jnp.float32),
            pltpu.VMEM_SHARED((npad, DG), jnp.float32),
            pltpu.SemaphoreType.DMA,
        ],
    )
    def sc_agg(
        feat_hbm, sidx_hbm, dst_hbm, z_hbm, ones_hbm,
        agg_out, deg_out,
        sidx_v, didx_v, rows_v, ones_v, acc_sh, sem,
    ):
        cid = lax.axis_index("c")
        sid = lax.axis_index("s")
        r0 = pl.multiple_of(sid * rpt, 8)
        pltpu.sync_copy(ones_hbm, ones_v)

        def zero_acc():
            for j in range(rpt // WB):
                pltpu.sync_copy(z_hbm, acc_sh.at[pl.ds(r0 + j * WB, WB)])

        def edge_loop(body):
            def step(t, carry):
                e0 = pl.multiple_of((sid * cw + t) * CHUNK, 8)
                body(e0)
                return carry
            lax.fori_loop(0, cw, step, 0)

        for gi in range(gpc):
            g = cid * gpc + gi
            zero_acc()
            plsc.subcore_barrier()

            def gather_scatter(e0, g=g):
                pltpu.sync_copy(sidx_hbm.at[g, pl.ds(e0, CHUNK)], sidx_v)
                pltpu.sync_copy(dst_hbm.at[pl.ds(e0, CHUNK)], didx_v)
                pltpu.async_copy(feat_hbm.at[sidx_v], rows_v, sem).wait()
                pltpu.sync_copy(rows_v, acc_sh.at[didx_v], add=True)

            edge_loop(gather_scatter)
            plsc.subcore_barrier()
            for j in range(rpt // WB):
                pltpu.sync_copy(
                    acc_sh.at[pl.ds(r0 + j * WB, WB)],
                    agg_out.at[pl.ds(r0 + j * WB, WB), pl.ds(g * DG, DG)],
                )
            plsc.subcore_barrier()

        @pl.when(cid == 0)
        def _():
            zero_acc()
            plsc.subcore_barrier()

            def deg_scatter(e0):
                pltpu.sync_copy(dst_hbm.at[pl.ds(e0, CHUNK)], didx_v)
                pltpu.sync_copy(ones_v, acc_sh.at[didx_v], add=True)

            edge_loop(deg_scatter)
            plsc.subcore_barrier()
            for j in range(rpt // WB):
                pltpu.sync_copy(
                    acc_sh.at[pl.ds(r0 + j * WB, WB)],
                    deg_out.at[pl.ds(r0 + j * WB, WB)],
                )

    return sc_agg(feat16, sidx8, dst, zeros, ones)


def _tc_finish(agg_p, deg_p, W, b2, *, n):
    def tc_body(agg_ref, deg_ref, w_ref, b_ref, out_ref):
        inv = 1.0 / jnp.maximum(deg_ref[:, 0:1], 1.0)
        h = agg_ref[...] * inv
        y = lax.dot_general(
            h, w_ref[...], (((1,), (1,)), ((), ())),
            preferred_element_type=jnp.float32,
        )
        out_ref[...] = y[:n] + b_ref[...]

    return pl.pallas_call(
        tc_body,
        out_shape=jax.ShapeDtypeStruct((n, D), jnp.float32),
    )(agg_p, deg_p, W, b2)


def kernel(feature, edge_index, W, b):
    n = feature.shape[0]
    e = edge_index.shape[1]
    npad = (-(n + 1) // (NS * WB)) * -(NS * WB) if n >= NS * WB else NS * WB
    cw = -(-e // (NS * CHUNK))
    e_pad = cw * NS * CHUNK

    src = edge_index[0].astype(jnp.int32)
    dst = edge_index[1].astype(jnp.int32)
    if e_pad != e:
        src = jnp.concatenate([src, jnp.zeros((e_pad - e,), jnp.int32)])
        dst = jnp.concatenate([dst, jnp.full((e_pad - e,), n, jnp.int32)])

    sidx8 = src[None, :] * G + jnp.arange(G, dtype=jnp.int32)[:, None]
    feat16 = feature.reshape(n * G, DG)
    zeros = jnp.zeros((WB, DG), jnp.float32)
    ones = jnp.ones((CHUNK, DG), jnp.float32)

    agg_p, deg_p = _sc_aggregate(feat16, sidx8, dst, zeros, ones,
                                 npad=npad, cw=cw)
    return _tc_finish(agg_p, deg_p, W, b.reshape(1, D), n=n)


# speedup vs baseline: 5.3795x; 3.1122x over previous
import functools

import jax
import jax.numpy as jnp
from jax import lax
from jax.experimental import pallas as pl
from jax.experimental.pallas import tpu as pltpu
from jax.experimental.pallas import tpu_sc as plsc

NC = 2
NS = 16
CHUNK = 128
D = 128
G = 8
DG = D // G
WB = 320


def _sc_aggregate(feat16, sidx3, dst2, zeros, ones, *, npad, cw):
    rpt = npad // NS
    gpc = G // NC
    mesh = plsc.VectorSubcoreMesh(
        core_axis_name="c", subcore_axis_name="s", num_cores=NC, num_subcores=NS
    )

    @functools.partial(
        pl.kernel,
        out_type=(
            jax.ShapeDtypeStruct((npad, D), jnp.float32),
            jax.ShapeDtypeStruct((npad, DG), jnp.float32),
        ),
        mesh=mesh,
        compiler_params=pltpu.CompilerParams(use_tc_tiling_on_sc=False),
        scratch_types=[
            pltpu.VMEM((cw, CHUNK), jnp.int32),
            pltpu.VMEM((cw, CHUNK), jnp.int32),
            pltpu.VMEM((2, CHUNK, DG), jnp.float32),
            pltpu.VMEM((CHUNK, DG), jnp.float32),
            pltpu.VMEM_SHARED((npad, DG), jnp.float32),
            pltpu.SemaphoreType.DMA((2,)),
        ],
    )
    def sc_agg(
        feat_hbm, sidx_hbm, dst_hbm, z_hbm, ones_hbm,
        agg_out, deg_out,
        sidx_v, didx_v, rows_v, ones_v, acc_sh, sem,
    ):
        cid = lax.axis_index("c")
        sid = lax.axis_index("s")
        r0 = pl.multiple_of(sid * rpt, 8)
        c0 = sid * cw
        pltpu.sync_copy(ones_hbm, ones_v)
        pltpu.sync_copy(dst_hbm.at[pl.ds(c0, cw)], didx_v)

        def zero_acc():
            for j in range(rpt // WB):
                pltpu.sync_copy(z_hbm, acc_sh.at[pl.ds(r0 + j * WB, WB)])

        def gather_issue(t, b):
            pltpu.async_copy(feat_hbm.at[sidx_v.at[t]], rows_v.at[b], sem.at[b])

        def gather_wait(b):
            pltpu.make_async_copy(
                feat_hbm.at[sidx_v.at[0]], rows_v.at[b], sem.at[b]
            ).wait()

        def scatter(t, b):
            pltpu.sync_copy(rows_v.at[b], acc_sh.at[didx_v.at[t]], add=True)

        for gi in range(gpc):
            g = cid * gpc + gi
            zero_acc()
            pltpu.sync_copy(sidx_hbm.at[g, pl.ds(c0, cw)], sidx_v)
            plsc.subcore_barrier()

            gather_issue(0, 0)

            def two_steps(i, carry):
                t0 = 2 * i

                @pl.when(t0 + 1 < cw)
                def _():
                    gather_issue(t0 + 1, 1)

                gather_wait(0)
                scatter(t0, 0)

                @pl.when(t0 + 1 < cw)
                def _():
                    @pl.when(t0 + 2 < cw)
                    def _():
                        gather_issue(t0 + 2, 0)

                    gather_wait(1)
                    scatter(t0 + 1, 1)

                return carry

            lax.fori_loop(0, (cw + 1) // 2, two_steps, 0)
            plsc.subcore_barrier()
            for j in range(rpt // WB):
                pltpu.sync_copy(
                    acc_sh.at[pl.ds(r0 + j * WB, WB)],
                    agg_out.at[pl.ds(r0 + j * WB, WB), pl.ds(g * DG, DG)],
                )
            plsc.subcore_barrier()

        @pl.when(cid == 0)
        def _():
            zero_acc()
            plsc.subcore_barrier()

            def deg_issue(t, b):
                pltpu.async_copy(
                    ones_v, acc_sh.at[didx_v.at[t]], sem.at[b], add=True
                )

            def deg_wait(t, b):
                pltpu.make_async_copy(
                    ones_v, acc_sh.at[didx_v.at[t]], sem.at[b]
                ).wait()

            deg_issue(0, 0)

            def deg_steps(i, carry):
                t0 = 2 * i

                @pl.when(t0 + 1 < cw)
                def _():
                    deg_issue(t0 + 1, 1)

                deg_wait(t0, 0)

                @pl.when(t0 + 1 < cw)
                def _():
                    @pl.when(t0 + 2 < cw)
                    def _():
                        deg_issue(t0 + 2, 0)

                    deg_wait(t0 + 1, 1)

                return carry

            lax.fori_loop(0, (cw + 1) // 2, deg_steps, 0)
            plsc.subcore_barrier()
            for j in range(rpt // WB):
                pltpu.sync_copy(
                    acc_sh.at[pl.ds(r0 + j * WB, WB)],
                    deg_out.at[pl.ds(r0 + j * WB, WB)],
                )

    return sc_agg(feat16, sidx3, dst2, zeros, ones)


def _tc_finish(agg_p, deg_p, W, b2, *, n):
    def tc_body(agg_ref, deg_ref, w_ref, b_ref, out_ref):
        inv = 1.0 / jnp.maximum(deg_ref[:, 0:1], 1.0)
        h = agg_ref[...] * inv
        y = lax.dot_general(
            h, w_ref[...], (((1,), (1,)), ((), ())),
            preferred_element_type=jnp.float32,
        )
        out_ref[...] = y[:n] + b_ref[...]

    return pl.pallas_call(
        tc_body,
        out_shape=jax.ShapeDtypeStruct((n, D), jnp.float32),
    )(agg_p, deg_p, W, b2)


def kernel(feature, edge_index, W, b):
    n = feature.shape[0]
    e = edge_index.shape[1]
    npad = (-(n + 1) // (NS * WB)) * -(NS * WB) if n >= NS * WB else NS * WB
    cw = -(-e // (NS * CHUNK))
    e_pad = cw * NS * CHUNK

    src = edge_index[0].astype(jnp.int32)
    dst = edge_index[1].astype(jnp.int32)
    if e_pad != e:
        src = jnp.concatenate([src, jnp.zeros((e_pad - e,), jnp.int32)])
        dst = jnp.concatenate([dst, jnp.full((e_pad - e,), n, jnp.int32)])

    nch = e_pad // CHUNK
    sidx3 = (src[None, :] * G + jnp.arange(G, dtype=jnp.int32)[:, None]
             ).reshape(G, nch, CHUNK)
    dst2 = dst.reshape(nch, CHUNK)
    feat16 = feature.reshape(n * G, DG)
    zeros = jnp.zeros((WB, DG), jnp.float32)
    ones = jnp.ones((CHUNK, DG), jnp.float32)

    agg_p, deg_p = _sc_aggregate(feat16, sidx3, dst2, zeros, ones,
                                 npad=npad, cw=cw)
    return _tc_finish(agg_p, deg_p, W, b.reshape(1, D), n=n)


# speedup vs baseline: 7.4686x; 1.3883x over previous
import functools

import jax
import jax.numpy as jnp
from jax import lax
from jax.experimental import pallas as pl
from jax.experimental.pallas import tpu as pltpu
from jax.experimental.pallas import tpu_sc as plsc

NC = 2
NS = 16
CHUNK = 128
D = 128
G = 8
DG = D // G
WB = 320


def _sc_aggregate(feat16, sidx3, dst2, zeros, ones, *, npad, cw):
    rpt = npad // NS
    gpc = G // NC
    mesh = plsc.VectorSubcoreMesh(
        core_axis_name="c", subcore_axis_name="s", num_cores=NC, num_subcores=NS
    )

    @functools.partial(
        pl.kernel,
        out_type=(
            jax.ShapeDtypeStruct((npad, D), jnp.float32),
            jax.ShapeDtypeStruct((npad, DG), jnp.float32),
        ),
        mesh=mesh,
        compiler_params=pltpu.CompilerParams(use_tc_tiling_on_sc=False),
        scratch_types=[
            pltpu.VMEM((cw, CHUNK), jnp.int32),
            pltpu.VMEM((cw, CHUNK), jnp.int32),
            pltpu.VMEM((4, CHUNK, DG), jnp.float32),
            pltpu.VMEM((CHUNK, DG), jnp.float32),
            pltpu.VMEM_SHARED((npad, DG), jnp.float32),
            pltpu.SemaphoreType.DMA((4,)),
            pltpu.SemaphoreType.DMA((4,)),
        ],
    )
    def sc_agg(
        feat_hbm, sidx_hbm, dst_hbm, z_hbm, ones_hbm,
        agg_out, deg_out,
        sidx_v, didx_v, rows_v, ones_v, acc_sh, sem, sem_s,
    ):
        cid = lax.axis_index("c")
        sid = lax.axis_index("s")
        r0 = pl.multiple_of(sid * rpt, 8)
        c0 = sid * cw
        pltpu.sync_copy(ones_hbm, ones_v)
        pltpu.sync_copy(dst_hbm.at[pl.ds(c0, cw)], didx_v)

        def zero_acc():
            for j in range(rpt // WB):
                pltpu.sync_copy(z_hbm, acc_sh.at[pl.ds(r0 + j * WB, WB)])

        def gather_issue(t, b):
            pltpu.async_copy(feat_hbm.at[sidx_v.at[t]], rows_v.at[b], sem.at[b])

        def gather_wait(b):
            pltpu.make_async_copy(
                feat_hbm.at[sidx_v.at[0]], rows_v.at[b], sem.at[b]
            ).wait()

        def scatter_issue(t, b):
            pltpu.async_copy(
                rows_v.at[b], acc_sh.at[didx_v.at[t]], sem_s.at[b], add=True
            )

        def scatter_wait(b):
            pltpu.make_async_copy(
                rows_v.at[b], acc_sh.at[didx_v.at[0]], sem_s.at[b]
            ).wait()

        for gi in range(gpc):
            g = cid * gpc + gi
            zero_acc()
            pltpu.sync_copy(sidx_hbm.at[g, pl.ds(c0, cw)], sidx_v)
            plsc.subcore_barrier()

            for t in range(3):
                gather_issue(t, t)

            def step(t, carry):
                q = lax.rem(t, 4)

                @pl.when(t + 3 < cw)
                def _():
                    qn = lax.rem(t + 3, 4)

                    @pl.when(t >= 1)
                    def _():
                        scatter_wait(qn)

                    gather_issue(t + 3, qn)

                gather_wait(q)
                scatter_issue(t, q)
                return carry

            lax.fori_loop(0, cw, step, 0)
            for r in range(max(0, cw - 4), cw):
                scatter_wait(r % 4)

            plsc.subcore_barrier()
            for j in range(rpt // WB):
                pltpu.sync_copy(
                    acc_sh.at[pl.ds(r0 + j * WB, WB)],
                    agg_out.at[pl.ds(r0 + j * WB, WB), pl.ds(g * DG, DG)],
                )
            plsc.subcore_barrier()

        @pl.when(cid == 0)
        def _():
            zero_acc()
            plsc.subcore_barrier()

            def deg_issue(t, b):
                pltpu.async_copy(
                    ones_v, acc_sh.at[didx_v.at[t]], sem.at[b], add=True
                )

            def deg_wait(t, b):
                pltpu.make_async_copy(
                    ones_v, acc_sh.at[didx_v.at[t]], sem.at[b]
                ).wait()

            deg_issue(0, 0)

            def deg_steps(i, carry):
                t0 = 2 * i

                @pl.when(t0 + 1 < cw)
                def _():
                    deg_issue(t0 + 1, 1)

                deg_wait(t0, 0)

                @pl.when(t0 + 1 < cw)
                def _():
                    @pl.when(t0 + 2 < cw)
                    def _():
                        deg_issue(t0 + 2, 0)

                    deg_wait(t0 + 1, 1)

                return carry

            lax.fori_loop(0, (cw + 1) // 2, deg_steps, 0)
            plsc.subcore_barrier()
            for j in range(rpt // WB):
                pltpu.sync_copy(
                    acc_sh.at[pl.ds(r0 + j * WB, WB)],
                    deg_out.at[pl.ds(r0 + j * WB, WB)],
                )

    return sc_agg(feat16, sidx3, dst2, zeros, ones)


def _tc_finish(agg_p, deg_p, W, b2, *, n):
    def tc_body(agg_ref, deg_ref, w_ref, b_ref, out_ref):
        inv = 1.0 / jnp.maximum(deg_ref[:, 0:1], 1.0)
        h = agg_ref[...] * inv
        y = lax.dot_general(
            h, w_ref[...], (((1,), (1,)), ((), ())),
            preferred_element_type=jnp.float32,
        )
        out_ref[...] = y[:n] + b_ref[...]

    return pl.pallas_call(
        tc_body,
        out_shape=jax.ShapeDtypeStruct((n, D), jnp.float32),
    )(agg_p, deg_p, W, b2)


def kernel(feature, edge_index, W, b):
    n = feature.shape[0]
    e = edge_index.shape[1]
    npad = (-(n + 1) // (NS * WB)) * -(NS * WB) if n >= NS * WB else NS * WB
    cw = -(-e // (NS * CHUNK))
    e_pad = cw * NS * CHUNK

    src = edge_index[0].astype(jnp.int32)
    dst = edge_index[1].astype(jnp.int32)
    if e_pad != e:
        src = jnp.concatenate([src, jnp.zeros((e_pad - e,), jnp.int32)])
        dst = jnp.concatenate([dst, jnp.full((e_pad - e,), n, jnp.int32)])

    nch = e_pad // CHUNK
    sidx3 = (src[None, :] * G + jnp.arange(G, dtype=jnp.int32)[:, None]
             ).reshape(G, nch, CHUNK)
    dst2 = dst.reshape(nch, CHUNK)
    feat16 = feature.reshape(n * G, DG)
    zeros = jnp.zeros((WB, DG), jnp.float32)
    ones = jnp.ones((CHUNK, DG), jnp.float32)

    agg_p, deg_p = _sc_aggregate(feat16, sidx3, dst2, zeros, ones,
                                 npad=npad, cw=cw)
    return _tc_finish(agg_p, deg_p, W, b.reshape(1, D), n=n)


# speedup vs baseline: 7.4770x; 1.0011x over previous
import functools

import jax
import jax.numpy as jnp
from jax import lax
from jax.experimental import pallas as pl
from jax.experimental.pallas import tpu as pltpu
from jax.experimental.pallas import tpu_sc as plsc

NC = 2
NS = 16
CHUNK = 128
D = 128
G = 8
DG = D // G
WB = 320


def _sc_aggregate(feat16, sidx3, dst2, zeros, ones, *, npad, cw):
    rpt = npad // NS
    gpc = G // NC
    mesh = plsc.VectorSubcoreMesh(
        core_axis_name="c", subcore_axis_name="s", num_cores=NC, num_subcores=NS
    )

    @functools.partial(
        pl.kernel,
        out_type=(
            jax.ShapeDtypeStruct((npad, D), jnp.float32),
            jax.ShapeDtypeStruct((npad, DG), jnp.float32),
        ),
        mesh=mesh,
        compiler_params=pltpu.CompilerParams(use_tc_tiling_on_sc=False),
        scratch_types=[
            pltpu.VMEM((cw, CHUNK), jnp.int32),
            pltpu.VMEM((cw, CHUNK), jnp.int32),
            pltpu.VMEM((4, CHUNK, DG), jnp.float32),
            pltpu.VMEM((CHUNK, DG), jnp.float32),
            pltpu.VMEM_SHARED((npad, DG), jnp.float32),
            pltpu.SemaphoreType.DMA((4,)),
            pltpu.SemaphoreType.DMA((4,)),
        ],
    )
    def sc_agg(
        feat_hbm, sidx_hbm, dst_hbm, z_hbm, ones_hbm,
        agg_out, deg_out,
        sidx_v, didx_v, rows_v, ones_v, acc_sh, sem, sem_s,
    ):
        cid = lax.axis_index("c")
        sid = lax.axis_index("s")
        r0 = pl.multiple_of(sid * rpt, 8)
        c0 = sid * cw
        pltpu.sync_copy(ones_hbm, ones_v)
        pltpu.sync_copy(dst_hbm.at[pl.ds(c0, cw)], didx_v)

        def zero_acc():
            for j in range(rpt // WB):
                pltpu.sync_copy(z_hbm, acc_sh.at[pl.ds(r0 + j * WB, WB)])

        def gather_issue(t, b):
            pltpu.async_copy(feat_hbm.at[sidx_v.at[t]], rows_v.at[b], sem.at[b])

        def gather_wait(b):
            pltpu.make_async_copy(
                feat_hbm.at[sidx_v.at[0]], rows_v.at[b], sem.at[b]
            ).wait()

        def scatter_issue(t, b):
            pltpu.async_copy(
                rows_v.at[b], acc_sh.at[didx_v.at[t]], sem_s.at[b], add=True
            )

        def scatter_wait(b):
            pltpu.make_async_copy(
                rows_v.at[b], acc_sh.at[didx_v.at[0]], sem_s.at[b]
            ).wait()

        for gi in range(gpc):
            g = cid * gpc + gi
            zero_acc()
            pltpu.sync_copy(sidx_hbm.at[g, pl.ds(c0, cw)], sidx_v)
            plsc.subcore_barrier()

            for t in range(3):
                gather_issue(t, t)

            def step(t, carry):
                q = lax.rem(t, 4)

                @pl.when(t + 3 < cw)
                def _():
                    qn = lax.rem(t + 3, 4)

                    @pl.when(t >= 1)
                    def _():
                        scatter_wait(qn)

                    gather_issue(t + 3, qn)

                gather_wait(q)
                scatter_issue(t, q)
                return carry

            lax.fori_loop(0, cw, step, 0)
            for r in range(max(0, cw - 4), cw):
                scatter_wait(r % 4)

            plsc.subcore_barrier()
            for j in range(rpt // WB):
                pltpu.sync_copy(
                    acc_sh.at[pl.ds(r0 + j * WB, WB)],
                    agg_out.at[pl.ds(r0 + j * WB, WB), pl.ds(g * DG, DG)],
                )
            plsc.subcore_barrier()

        @pl.when(cid == 0)
        def _():
            zero_acc()
            plsc.subcore_barrier()

            def deg_issue(t, b):
                pltpu.async_copy(
                    ones_v, acc_sh.at[didx_v.at[t]], sem.at[b], add=True
                )

            def deg_wait(b):
                pltpu.make_async_copy(
                    ones_v, acc_sh.at[didx_v.at[0]], sem.at[b]
                ).wait()

            for t in range(min(4, cw)):
                deg_issue(t, t)

            def deg_step(t, carry):
                q = lax.rem(t, 4)
                deg_wait(q)

                @pl.when(t + 4 < cw)
                def _():
                    deg_issue(t + 4, q)

                return carry

            lax.fori_loop(0, cw, deg_step, 0)
            plsc.subcore_barrier()
            for j in range(rpt // WB):
                pltpu.sync_copy(
                    acc_sh.at[pl.ds(r0 + j * WB, WB)],
                    deg_out.at[pl.ds(r0 + j * WB, WB)],
                )

    return sc_agg(feat16, sidx3, dst2, zeros, ones)


def _tc_finish(agg_p, deg_p, W, b2, *, n):
    def tc_body(agg_ref, deg_ref, w_ref, b_ref, out_ref):
        inv = 1.0 / jnp.maximum(deg_ref[:, 0:1], 1.0)
        h = agg_ref[...] * inv
        y = lax.dot_general(
            h, w_ref[...], (((1,), (1,)), ((), ())),
            preferred_element_type=jnp.float32,
        )
        out_ref[...] = y[:n] + b_ref[...]

    return pl.pallas_call(
        tc_body,
        out_shape=jax.ShapeDtypeStruct((n, D), jnp.float32),
    )(agg_p, deg_p, W, b2)


def kernel(feature, edge_index, W, b):
    n = feature.shape[0]
    e = edge_index.shape[1]
    npad = (-(n + 1) // (NS * WB)) * -(NS * WB) if n >= NS * WB else NS * WB
    cw = -(-e // (NS * CHUNK))
    e_pad = cw * NS * CHUNK

    src = edge_index[0].astype(jnp.int32)
    dst = edge_index[1].astype(jnp.int32)
    if e_pad != e:
        src = jnp.concatenate([src, jnp.zeros((e_pad - e,), jnp.int32)])
        dst = jnp.concatenate([dst, jnp.full((e_pad - e,), n, jnp.int32)])

    nch = e_pad // CHUNK
    sidx3 = (src[None, :] * G + jnp.arange(G, dtype=jnp.int32)[:, None]
             ).reshape(G, nch, CHUNK)
    dst2 = dst.reshape(nch, CHUNK)
    feat16 = feature.reshape(n * G, DG)
    zeros = jnp.zeros((WB, DG), jnp.float32)
    ones = jnp.ones((CHUNK, DG), jnp.float32)

    agg_p, deg_p = _sc_aggregate(feat16, sidx3, dst2, zeros, ones,
                                 npad=npad, cw=cw)
    return _tc_finish(agg_p, deg_p, W, b.reshape(1, D), n=n)
